# Initial kernel scaffold; baseline (speedup 1.0000x reference)
import jax
import jax.numpy as jnp
from jax.experimental import pallas as pl


def kernel(x, W_enc_kernel, W_dec, b_enc, b_dec):
    raise NotImplementedError("write your pallas kernel here")


# speedup vs baseline: 5.0959x; 5.0959x over previous
import functools

import jax
import jax.numpy as jnp
from jax.experimental import pallas as pl
from jax.experimental.pallas import tpu as pltpu

D_IN_ = 256
D_SAE_ = 4096
T_ = 32
K_ = 32
B_ = 4
M_ = B_ * T_
KC_ = T_ * D_IN_

BK_ = 1024
BS_ = 1024
NK_ = KC_ // BK_
NS_ = D_SAE_ // BS_
OPB_ = BK_ // D_IN_


def _encode_body(xcat_ref, w_ref, out_ref, xbig_ref):
    s = pl.program_id(0)
    k = pl.program_id(1)

    @pl.when((s == 0) & (k == 0))
    def _build():
        for o in range(T_):
            pieces = [xcat_ref[b * 2 * T_ + T_ - o: b * 2 * T_ + 2 * T_ - o, :]
                      for b in range(B_)]
            blk = o // OPB_
            col = (o % OPB_) * D_IN_
            xbig_ref[blk, :, col:col + D_IN_] = jnp.concatenate(pieces, axis=0)

    acc = jnp.dot(xbig_ref[k], w_ref[...], preferred_element_type=jnp.float32)

    @pl.when(k == 0)
    def _init():
        out_ref[...] = acc

    @pl.when(k > 0)
    def _acc():
        out_ref[...] += acc


def _select_decode_body(pre_ref, x_ref, wdec_ref, benc_ref, bdec_ref,
                        z_ref, xhat_ref, loss_ref, work_ref):
    pre = pre_ref[...] + benc_ref[...]
    work_ref[...] = pre

    def body(i, _):
        w = work_ref[...]
        m = jnp.max(w, axis=1, keepdims=True)
        work_ref[...] = jnp.where(w >= m, -jnp.inf, w)
        return 0

    jax.lax.fori_loop(0, K_ - 1, body, 0)
    thr = jnp.max(work_ref[...], axis=1, keepdims=True)
    z = jnp.where(pre >= thr, jnp.maximum(pre, 0.0), 0.0)
    z_ref[...] = z
    xhat = (jnp.dot(z, wdec_ref[...], preferred_element_type=jnp.float32)
            + bdec_ref[...])
    xhat_ref[...] = xhat
    d = xhat - x_ref[...]
    loss_ref[0, 0] = jnp.sum(d * d) * (1.0 / M_)


@jax.jit
def kernel(x, W_enc_kernel, W_dec, b_enc, b_dec):
    xcat = jnp.pad(x, ((0, 0), (T_, 0), (0, 0))).reshape(B_ * 2 * T_, D_IN_)
    wbig = W_enc_kernel.reshape(KC_, D_SAE_)

    pre = pl.pallas_call(
        _encode_body,
        grid=(NS_, NK_),
        in_specs=[
            pl.BlockSpec((B_ * 2 * T_, D_IN_), lambda s, k: (0, 0)),
            pl.BlockSpec((BK_, BS_), lambda s, k: (k, s)),
        ],
        out_specs=pl.BlockSpec((M_, BS_), lambda s, k: (0, s)),
        out_shape=jax.ShapeDtypeStruct((M_, D_SAE_), jnp.float32),
        scratch_shapes=[pltpu.VMEM((NK_, M_, BK_), jnp.float32)],
    )(xcat, wbig)

    x2 = x.reshape(M_, D_IN_)
    z2, xhat2, loss2 = pl.pallas_call(
        _select_decode_body,
        in_specs=[
            pl.BlockSpec((M_, D_SAE_), lambda: (0, 0)),
            pl.BlockSpec((M_, D_IN_), lambda: (0, 0)),
            pl.BlockSpec((D_SAE_, D_IN_), lambda: (0, 0)),
            pl.BlockSpec((1, D_SAE_), lambda: (0, 0)),
            pl.BlockSpec((1, D_IN_), lambda: (0, 0)),
        ],
        out_specs=[
            pl.BlockSpec((M_, D_SAE_), lambda: (0, 0)),
            pl.BlockSpec((M_, D_IN_), lambda: (0, 0)),
            pl.BlockSpec(memory_space=pltpu.SMEM),
        ],
        out_shape=[
            jax.ShapeDtypeStruct((M_, D_SAE_), jnp.float32),
            jax.ShapeDtypeStruct((M_, D_IN_), jnp.float32),
            jax.ShapeDtypeStruct((1, 1), jnp.float32),
        ],
        scratch_shapes=[pltpu.VMEM((M_, D_SAE_), jnp.float32)],
    )(pre, x2, W_dec, b_enc.reshape(1, D_SAE_), b_dec.reshape(1, D_IN_))

    z = z2.reshape(B_, T_, D_SAE_)
    x_hat = xhat2.reshape(B_, T_, D_IN_)
    loss = loss2[0, 0]
    return (loss, x_hat, z)
